# Initial kernel scaffold; baseline (speedup 1.0000x reference)
import jax
import jax.numpy as jnp
from jax.experimental import pallas as pl


def kernel(x, edge_index, W1, b1, W2, b2):
    raise NotImplementedError("write your pallas kernel here")


# speedup vs baseline: 13.0758x; 13.0758x over previous
import functools

import jax
import jax.numpy as jnp
from jax import lax
from jax.experimental import pallas as pl
from jax.experimental.pallas import tpu as pltpu
from jax.experimental.pallas import tpu_sc as plsc

NN = 10000
NP = 10240
EE = 320000
DD = 128

NC = 2
NS = 16
NW = NC * NS
EPW = EE // NW
CHUNK = 80
NCHUNK = EPW // CHUNK
RPT = NP // NS

_mesh = plsc.VectorSubcoreMesh(core_axis_name="c", subcore_axis_name="s")


@functools.partial(
    pl.kernel,
    out_type=jax.ShapeDtypeStruct((NC, NP, 16), jnp.float32),
    mesh=_mesh,
    scratch_types=[
        pltpu.VMEM((CHUNK,), jnp.int32),
        pltpu.VMEM((CHUNK, 16), jnp.float32),
        pltpu.VMEM_SHARED((NP, 16), jnp.float32),
    ],
)
def _deg_kernel(dst_hbm, ones_hbm, zeros_hbm, out_hbm, idx_v, ones_v, deg_sh):
    c = lax.axis_index("c")
    s = lax.axis_index("s")
    wid = s * NC + c
    pltpu.sync_copy(zeros_hbm, deg_sh.at[pl.ds(s * RPT, RPT)])
    pltpu.sync_copy(ones_hbm, ones_v)
    plsc.subcore_barrier()

    def body(i, carry):
        base = wid * EPW + i * CHUNK
        pltpu.sync_copy(dst_hbm.at[pl.ds(base, CHUNK)], idx_v)
        pltpu.sync_copy(ones_v, deg_sh.at[idx_v], add=True)
        return carry

    lax.fori_loop(0, NCHUNK, body, 0)
    plsc.subcore_barrier()
    pltpu.sync_copy(deg_sh.at[pl.ds(s * RPT, RPT)],
                    out_hbm.at[c, pl.ds(s * RPT, RPT)])


@functools.partial(
    pl.kernel,
    out_type=jax.ShapeDtypeStruct((NC, NP, DD), jnp.float32),
    mesh=_mesh,
    scratch_types=[
        pltpu.VMEM((CHUNK,), jnp.int32),
        pltpu.VMEM((CHUNK,), jnp.int32),
        pltpu.VMEM((CHUNK, DD), jnp.float32),
        pltpu.VMEM_SHARED((NP, DD), jnp.float32),
        pltpu.SemaphoreType.DMA,
    ],
)
def _agg_kernel(g_hbm, src_hbm, dst_hbm, zeros_hbm, out_hbm,
                sidx, didx, rows, acc_sh, sem):
    c = lax.axis_index("c")
    s = lax.axis_index("s")
    wid = s * NC + c
    pltpu.sync_copy(zeros_hbm, acc_sh.at[pl.ds(s * RPT, RPT)])
    plsc.subcore_barrier()

    def body(i, carry):
        base = wid * EPW + i * CHUNK
        pltpu.sync_copy(src_hbm.at[pl.ds(base, CHUNK)], sidx)
        pltpu.sync_copy(dst_hbm.at[pl.ds(base, CHUNK)], didx)
        pltpu.async_copy(g_hbm.at[sidx], rows, sem).wait()
        pltpu.sync_copy(rows, acc_sh.at[didx], add=True)
        return carry

    lax.fori_loop(0, NCHUNK, body, 0)
    plsc.subcore_barrier()
    pltpu.sync_copy(acc_sh.at[pl.ds(s * RPT, RPT)],
                    out_hbm.at[c, pl.ds(s * RPT, RPT)])


_ROWS_BLK = 1024
_GRID = NP // _ROWS_BLK


def _dis_from(deg_ref):
    deg = 1.0 + deg_ref[0, :, 0:1] + deg_ref[1, :, 0:1]
    return lax.rsqrt(deg)


def _tc1_body(deg_ref, x_ref, w_ref, g_ref):
    dis = _dis_from(deg_ref)
    h = jnp.dot(x_ref[...], w_ref[...], preferred_element_type=jnp.float32)
    g_ref[...] = dis * h


def _tc2_body(deg_ref, acc_ref, g1_ref, b1_ref, w2_ref, g2_ref):
    dis = _dis_from(deg_ref)
    h = dis * (acc_ref[0] + acc_ref[1] + g1_ref[...]) + b1_ref[...]
    h = jnp.maximum(h, 0.0)
    g2_ref[...] = dis * jnp.dot(h, w2_ref[...],
                                preferred_element_type=jnp.float32)


def _tc3_body(deg_ref, acc_ref, g2_ref, b2_ref, out_ref):
    dis = _dis_from(deg_ref)
    t = dis * (acc_ref[0] + acc_ref[1] + g2_ref[...]) + b2_ref[...]
    m = jnp.max(t, axis=1, keepdims=True)
    lse = jnp.log(jnp.sum(jnp.exp(t - m), axis=1, keepdims=True)) + m
    out_ref[...] = t - lse


_deg_spec = pl.BlockSpec((NC, _ROWS_BLK, 16), lambda i: (0, i, 0))
_acc_spec = pl.BlockSpec((NC, _ROWS_BLK, DD), lambda i: (0, i, 0))
_row_spec = pl.BlockSpec((_ROWS_BLK, DD), lambda i: (i, 0))
_mat_spec = pl.BlockSpec((DD, DD), lambda i: (0, 0))
_vec_spec = pl.BlockSpec((1, DD), lambda i: (0, 0))
_out_row = jax.ShapeDtypeStruct((NP, DD), jnp.float32)

_tc1 = pl.pallas_call(
    _tc1_body, grid=(_GRID,),
    in_specs=[_deg_spec, _row_spec, _mat_spec],
    out_specs=_row_spec, out_shape=_out_row)

_tc2 = pl.pallas_call(
    _tc2_body, grid=(_GRID,),
    in_specs=[_deg_spec, _acc_spec, _row_spec, _vec_spec, _mat_spec],
    out_specs=_row_spec, out_shape=_out_row)

_tc3 = pl.pallas_call(
    _tc3_body, grid=(_GRID,),
    in_specs=[_deg_spec, _acc_spec, _row_spec, _vec_spec],
    out_specs=_row_spec, out_shape=_out_row)


def kernel(x, edge_index, W1, b1, W2, b2):
    src = edge_index[0].astype(jnp.int32)
    dst = edge_index[1].astype(jnp.int32)
    xp = jnp.pad(x, ((0, NP - NN), (0, 0)))
    ones16 = jnp.ones((CHUNK, 16), jnp.float32)
    zeros16 = jnp.zeros((RPT, 16), jnp.float32)
    zerosD = jnp.zeros((RPT, DD), jnp.float32)
    b1r = b1.reshape(1, DD)
    b2r = b2.reshape(1, DD)

    deg = _deg_kernel(dst, ones16, zeros16)
    g1 = _tc1(deg, xp, W1)
    acc1 = _agg_kernel(g1, src, dst, zerosD)
    g2 = _tc2(deg, acc1, g1, b1r, W2)
    acc2 = _agg_kernel(g2, src, dst, zerosD)
    return _tc3(deg, acc2, g2, b2r)[:NN]
